# baseline (speedup 1.0000x reference)
import jax
import jax.numpy as jnp
from jax.experimental import pallas as pl
from jax.experimental.pallas import tpu as pltpu

HIDDEN = 2048
N_EXPERTS = 64
BM = 2048
NBUF = 2


def _router_kernel(hs_hbm, w_ref, out_ref, buf, in_sem):
    nsteps = hs_hbm.shape[0] // BM

    def in_copy(step, slot):
        return pltpu.make_async_copy(
            hs_hbm.at[pl.ds(step * BM, BM)], buf.at[slot], in_sem.at[slot]
        )

    in_copy(0, 0).start()
    w = w_ref[...]
    for i in range(nsteps):
        slot = i % NBUF
        if i + 1 < nsteps:
            in_copy(i + 1, (i + 1) % NBUF).start()
        in_copy(i, slot).wait()
        out_ref[pl.ds(i * BM, BM)] = jax.lax.dot_general(
            buf[slot],
            w,
            dimension_numbers=(((1,), (1,)), ((), ())),
            preferred_element_type=jnp.float32,
        )


def kernel(hidden_states, weight):
    hs = hidden_states.reshape(-1, HIDDEN).astype(jnp.float32)
    w = weight.astype(jnp.float32)
    m = hs.shape[0]
    return pl.pallas_call(
        _router_kernel,
        in_specs=[
            pl.BlockSpec(memory_space=pltpu.HBM),
            pl.BlockSpec(memory_space=pltpu.VMEM),
        ],
        out_specs=pl.BlockSpec(memory_space=pltpu.VMEM),
        out_shape=jax.ShapeDtypeStruct((m, N_EXPERTS), jnp.float32),
        scratch_shapes=[
            pltpu.VMEM((NBUF, BM, HIDDEN), jnp.float32),
            pltpu.SemaphoreType.DMA((NBUF,)),
        ],
    )(hs, w)
